# baseline (speedup 1.0000x reference)
import jax
import jax.numpy as jnp
from jax import lax
from jax.experimental import pallas as pl
from jax.experimental.pallas import tpu as pltpu
from jax.experimental.pallas import tpu_sc as plsc

B = 4096
N = 10
LW = 20
D = 32
SEG = N + 1
ROWS = SEG * LW
CHUNK = 112
PADROWS = 2 * CHUNK
NC, NS = 2, 16
NW = NC * NS
BPW = B // NW
VL = 16


TBLK = 7812
TAILW = 64
BLK_PER_W = 246


def _sc_transpose(tt_hbm, tail_hbm, flat_hbm, tv, ov0, ov1, sem_i, sem_o):
    wid = lax.axis_index("s") * NC + lax.axis_index("c")
    start = wid * BLK_PER_W
    base_idx = lax.iota(jnp.int32, VL) * D
    ov = (ov0, ov1)

    def issue_in(vb, k):
        @pl.when(vb < TBLK)
        def _():
            pltpu.async_copy(tt_hbm.at[:, pl.ds(vb * 128, 128)],
                             tv.at[k], sem_i.at[k])

    def wait_in(vb, k):
        @pl.when(vb < TBLK)
        def _():
            pltpu.make_async_copy(tt_hbm.at[:, pl.ds(0, 128)],
                                  tv.at[k], sem_i.at[k]).wait()

    def transpose_buf(k):
        for d in range(D):
            for lb in range(8):
                v = tv[k, d, pl.ds(lb * VL, VL)]
                plsc.store_scatter(
                    ov[k], [base_idx + (lb * VL * D + d)], v)

    issue_in(start, 0)

    @pl.loop(0, BLK_PER_W, step=2)
    def _(j):
        for k in range(2):
            vb = start + j + k
            wait_in(vb, k)
            @pl.when(j + k + 1 < BLK_PER_W)
            def _():
                issue_in(vb + 1, 1 - k)
            @pl.when((j + k >= 2) & (vb - 2 < TBLK))
            def _():
                pltpu.make_async_copy(ov[k],
                                      flat_hbm.at[pl.ds(0, 128 * D)],
                                      sem_o.at[k]).wait()

            @pl.when(vb < TBLK)
            def _():
                transpose_buf(k)
                pltpu.async_copy(ov[k],
                                 flat_hbm.at[pl.ds(vb * 128 * D, 128 * D)],
                                 sem_o.at[k])

    for k in range(2):
        vb_last = start + BLK_PER_W - 2 + k
        @pl.when(vb_last < TBLK)
        def _():
            pltpu.make_async_copy(ov[k],
                                  flat_hbm.at[pl.ds(0, 128 * D)],
                                  sem_o.at[k]).wait()

    @pl.when(wid == NW - 1)
    def _():
        pltpu.sync_copy(tail_hbm, tv.at[0])
        transpose_buf(0)
        pltpu.sync_copy(ov0.at[pl.ds(0, TAILW * D)],
                        flat_hbm.at[pl.ds(TBLK * 128 * D, TAILW * D)])


def _sc_body(idx_hbm, table_hbm, out_hbm, idx_v, rows_v, out_v, sem_a, sem_b):
    wid = lax.axis_index("s") * NC + lax.axis_index("c")
    base = wid * BPW

    pltpu.sync_copy(idx_hbm.at[pl.ds(base, BPW)], idx_v)

    sems = (sem_a, sem_b)

    def issue(b, buf):
        pltpu.async_copy(table_hbm.at[idx_v.at[b, 0]],
                         rows_v.at[buf, pl.ds(0, CHUNK)], sems[buf])
        pltpu.async_copy(table_hbm.at[idx_v.at[b, 1]],
                         rows_v.at[buf, pl.ds(CHUNK, CHUNK)], sems[buf])

    def wait(b, buf):
        for c in range(2):
            pltpu.make_async_copy(table_hbm.at[idx_v.at[b, c]],
                                  rows_v.at[buf, pl.ds(c * CHUNK, CHUNK)],
                                  sems[buf]).wait()

    zeros = jnp.zeros((VL,), jnp.float32)

    def compute(b, buf):
        for s in range(SEG):
            acc = [zeros] * (6 if s == 0 else 4)
            for j in range(LW):
                r = s * LW + j
                v0 = rows_v[buf, r, pl.ds(0, VL)]
                v1 = rows_v[buf, r, pl.ds(VL, VL)]
                acc[0] = acc[0] + v0
                acc[1] = acc[1] + v1
                acc[2] = acc[2] + jnp.where(v0 != 0.0, 1.0, 0.0)
                acc[3] = acc[3] + jnp.where(v1 != 0.0, 1.0, 0.0)
                if s == 0:
                    acc[4] = acc[4] + v0 * v0
                    acc[5] = acc[5] + v1 * v1
            out_v[s, b, pl.ds(0, VL)] = acc[0] / acc[2]
            out_v[s, b, pl.ds(VL, VL)] = acc[1] / acc[3]
            if s == 0:
                out_v[11, b, pl.ds(0, VL)] = acc[0]
                out_v[11, b, pl.ds(VL, VL)] = acc[1]
                out_v[12, b, pl.ds(0, VL)] = acc[4]
                out_v[12, b, pl.ds(VL, VL)] = acc[5]

    issue(0, 0)

    @pl.loop(0, BPW, step=2)
    def _(i):
        for k in range(2):
            b = i + k
            wait(b, k)
            if k == 0:
                issue(b + 1, 1)
            else:
                @pl.when(b + 1 < BPW)
                def _():
                    issue(b + 1, 0)
            compute(b, k)

    for s in range(13):
        pltpu.sync_copy(out_v.at[s], out_hbm.at[s, pl.ds(base, BPW)])


def _tc_finish(sc_ref, words_ref, marg_ref, out_ref):
    i = pl.program_id(0)
    c = sc_ref[0]
    s1 = sc_ref[11]
    ssq = sc_ref[12]
    cnt2 = jnp.sum((words_ref[...] != 0).astype(jnp.float32), axis=1,
                   keepdims=True)
    cnorm = jnp.sum(c * c, axis=1, keepdims=True)
    cdot = jnp.sum(c * s1, axis=1, keepdims=True)
    s2 = jnp.sum(ssq, axis=1, keepdims=True)
    pos = 0.5 * (cnt2 * (cnorm + 1e-9) - 2.0 * cdot + s2) / cnt2

    marg = marg_ref[...]
    acc = jnp.zeros_like(pos)
    for n in range(N):
        cn = sc_ref[1 + n]
        d2 = jnp.sum((c - cn) ** 2, axis=1, keepdims=True)
        t = jnp.maximum(marg[:, n:n + 1] - jnp.sqrt(d2 + 1e-9), 0.0)
        acc += t * t
    neg = 0.5 * acc / float(N)

    @pl.when(i == 0)
    def _():
        out_ref[...] = jnp.zeros((1, 1), jnp.float32)
    out_ref[...] += jnp.sum(pos + neg, keepdims=True)

    @pl.when(i == pl.num_programs(0) - 1)
    def _():
        out_ref[...] = out_ref[...] / float(B)


@jax.jit
def kernel(syn_words, neg_words, margins, table):
    syn_words = syn_words.astype(jnp.int32)
    neg_words = neg_words.astype(jnp.int32)
    idx = jnp.concatenate(
        [syn_words[:, None, :], neg_words], axis=1).reshape(B, ROWS)
    idx = jnp.concatenate(
        [idx, jnp.zeros((B, PADROWS - ROWS), jnp.int32)], axis=1)
    idx = idx.reshape(B, 2, CHUNK)

    mesh = plsc.VectorSubcoreMesh(
        core_axis_name="c", subcore_axis_name="s",
        num_cores=NC, num_subcores=NS)

    VOCAB = table.shape[0]
    tail = jnp.concatenate(
        [jnp.swapaxes(table[TBLK * 128:], 0, 1),
         jnp.zeros((D, 128 - TAILW), jnp.float32)], axis=1)
    flat = pl.kernel(
        _sc_transpose,
        out_type=jax.ShapeDtypeStruct((VOCAB * D,), jnp.float32),
        mesh=mesh,
        scratch_types=[
            pltpu.VMEM((2, D, 128), jnp.float32),
            pltpu.VMEM((128 * D,), jnp.float32),
            pltpu.VMEM((128 * D,), jnp.float32),
            pltpu.SemaphoreType.DMA((2,)),
            pltpu.SemaphoreType.DMA((2,)),
        ],
        compiler_params=pltpu.CompilerParams(needs_layout_passes=False),
    )(jnp.swapaxes(table, 0, 1), tail)
    table = flat.reshape(VOCAB, D)

    sc_out = pl.kernel(
        _sc_body,
        out_type=jax.ShapeDtypeStruct((13, B, D), jnp.float32),
        mesh=mesh,
        scratch_types=[
            pltpu.VMEM((BPW, 2, CHUNK), jnp.int32),
            pltpu.VMEM((2, PADROWS, D), jnp.float32),
            pltpu.VMEM((13, BPW, D), jnp.float32),
            pltpu.SemaphoreType.DMA,
            pltpu.SemaphoreType.DMA,
        ],
        compiler_params=pltpu.CompilerParams(use_tc_tiling_on_sc=False),
    )(idx, table)

    R = 512
    loss = pl.pallas_call(
        _tc_finish,
        grid=(B // R,),
        in_specs=[
            pl.BlockSpec((13, R, D), lambda i: (0, i, 0)),
            pl.BlockSpec((R, LW), lambda i: (i, 0)),
            pl.BlockSpec((R, N), lambda i: (i, 0)),
        ],
        out_specs=pl.BlockSpec((1, 1), lambda i: (0, 0)),
        out_shape=jax.ShapeDtypeStruct((1, 1), jnp.float32),
    )(sc_out, syn_words, margins)
    return loss[0, 0]


# speedup vs baseline: 1.3079x; 1.3079x over previous
import jax
import jax.numpy as jnp
from jax import lax
from jax.experimental import pallas as pl
from jax.experimental.pallas import tpu as pltpu
from jax.experimental.pallas import tpu_sc as plsc

B = 4096
N = 10
LW = 20
D = 32
SEG = N + 1
ROWS = SEG * LW
CHUNK = 112
PADROWS = 2 * CHUNK
NC, NS = 2, 16
NW = NC * NS
BPW = B // NW
VL = 16


TBLK = 7812
TAILW = 64
BLK_PER_W = 246


def _sc_transpose(tt_hbm, tail_hbm, flat_hbm, tv0, tv1, ov0, ov1,
                  sem_i, sem_o):
    wid = lax.axis_index("s") * NC + lax.axis_index("c")
    start = wid * BLK_PER_W
    tv = (tv0, tv1)
    ov = (ov0, ov1)
    i0 = lax.iota(jnp.int32, VL)
    i1 = i0 + VL

    def issue_in(vb, k):
        @pl.when(vb < TBLK)
        def _():
            pltpu.async_copy(tt_hbm.at[:, pl.ds(vb * 128, 128)],
                             tv[k], sem_i.at[k])

    def wait_in(vb, k):
        @pl.when(vb < TBLK)
        def _():
            pltpu.make_async_copy(tt_hbm.at[:, pl.ds(0, 128)],
                                  tv[k], sem_i.at[k]).wait()

    def transpose_buf(k):
        @plsc.parallel_loop(0, 128, unroll=8)
        def _(l):
            c = jnp.full((VL,), 1, jnp.int32) * l
            ov[k][pl.ds(l * D, VL)] = plsc.load_gather(tv[k], [i0, c])
            ov[k][pl.ds(l * D + VL, VL)] = plsc.load_gather(tv[k], [i1, c])

    issue_in(start, 0)

    @pl.loop(0, BLK_PER_W, step=2)
    def _(j):
        for k in range(2):
            vb = start + j + k
            wait_in(vb, k)
            @pl.when(j + k + 1 < BLK_PER_W)
            def _():
                issue_in(vb + 1, 1 - k)
            @pl.when((j + k >= 2) & (vb - 2 < TBLK))
            def _():
                pltpu.make_async_copy(ov[k],
                                      flat_hbm.at[pl.ds(0, 128 * D)],
                                      sem_o.at[k]).wait()

            @pl.when(vb < TBLK)
            def _():
                transpose_buf(k)
                pltpu.async_copy(ov[k],
                                 flat_hbm.at[pl.ds(vb * 128 * D, 128 * D)],
                                 sem_o.at[k])

    for k in range(2):
        vb_last = start + BLK_PER_W - 2 + k
        @pl.when(vb_last < TBLK)
        def _():
            pltpu.make_async_copy(ov[k],
                                  flat_hbm.at[pl.ds(0, 128 * D)],
                                  sem_o.at[k]).wait()

    @pl.when(wid == NW - 1)
    def _():
        pltpu.sync_copy(tail_hbm, tv0)
        transpose_buf(0)
        pltpu.sync_copy(ov0.at[pl.ds(0, TAILW * D)],
                        flat_hbm.at[pl.ds(TBLK * 128 * D, TAILW * D)])


def _sc_body(idx_hbm, table_hbm, out_hbm, idx_v, rows_v, out_v, sem_a, sem_b):
    wid = lax.axis_index("s") * NC + lax.axis_index("c")
    base = wid * BPW

    pltpu.sync_copy(idx_hbm.at[pl.ds(base, BPW)], idx_v)

    sems = (sem_a, sem_b)

    def issue(b, buf):
        pltpu.async_copy(table_hbm.at[idx_v.at[b, 0]],
                         rows_v.at[buf, pl.ds(0, CHUNK)], sems[buf])
        pltpu.async_copy(table_hbm.at[idx_v.at[b, 1]],
                         rows_v.at[buf, pl.ds(CHUNK, CHUNK)], sems[buf])

    def wait(b, buf):
        for c in range(2):
            pltpu.make_async_copy(table_hbm.at[idx_v.at[b, c]],
                                  rows_v.at[buf, pl.ds(c * CHUNK, CHUNK)],
                                  sems[buf]).wait()

    zeros = jnp.zeros((VL,), jnp.float32)

    def compute(b, buf):
        for s in range(SEG):
            acc = [zeros] * (6 if s == 0 else 4)
            for j in range(LW):
                r = s * LW + j
                v0 = rows_v[buf, r, pl.ds(0, VL)]
                v1 = rows_v[buf, r, pl.ds(VL, VL)]
                acc[0] = acc[0] + v0
                acc[1] = acc[1] + v1
                acc[2] = acc[2] + jnp.where(v0 != 0.0, 1.0, 0.0)
                acc[3] = acc[3] + jnp.where(v1 != 0.0, 1.0, 0.0)
                if s == 0:
                    acc[4] = acc[4] + v0 * v0
                    acc[5] = acc[5] + v1 * v1
            out_v[s, b, pl.ds(0, VL)] = acc[0] / acc[2]
            out_v[s, b, pl.ds(VL, VL)] = acc[1] / acc[3]
            if s == 0:
                out_v[11, b, pl.ds(0, VL)] = acc[0]
                out_v[11, b, pl.ds(VL, VL)] = acc[1]
                out_v[12, b, pl.ds(0, VL)] = acc[4]
                out_v[12, b, pl.ds(VL, VL)] = acc[5]

    issue(0, 0)

    @pl.loop(0, BPW, step=2)
    def _(i):
        for k in range(2):
            b = i + k
            wait(b, k)
            if k == 0:
                issue(b + 1, 1)
            else:
                @pl.when(b + 1 < BPW)
                def _():
                    issue(b + 1, 0)
            compute(b, k)

    for s in range(13):
        pltpu.sync_copy(out_v.at[s], out_hbm.at[s, pl.ds(base, BPW)])


def _tc_finish(sc_ref, words_ref, marg_ref, out_ref):
    i = pl.program_id(0)
    c = sc_ref[0]
    s1 = sc_ref[11]
    ssq = sc_ref[12]
    cnt2 = jnp.sum((words_ref[...] != 0).astype(jnp.float32), axis=1,
                   keepdims=True)
    cnorm = jnp.sum(c * c, axis=1, keepdims=True)
    cdot = jnp.sum(c * s1, axis=1, keepdims=True)
    s2 = jnp.sum(ssq, axis=1, keepdims=True)
    pos = 0.5 * (cnt2 * (cnorm + 1e-9) - 2.0 * cdot + s2) / cnt2

    marg = marg_ref[...]
    acc = jnp.zeros_like(pos)
    for n in range(N):
        cn = sc_ref[1 + n]
        d2 = jnp.sum((c - cn) ** 2, axis=1, keepdims=True)
        t = jnp.maximum(marg[:, n:n + 1] - jnp.sqrt(d2 + 1e-9), 0.0)
        acc += t * t
    neg = 0.5 * acc / float(N)

    @pl.when(i == 0)
    def _():
        out_ref[...] = jnp.zeros((1, 1), jnp.float32)
    out_ref[...] += jnp.sum(pos + neg, keepdims=True)

    @pl.when(i == pl.num_programs(0) - 1)
    def _():
        out_ref[...] = out_ref[...] / float(B)


@jax.jit
def kernel(syn_words, neg_words, margins, table):
    syn_words = syn_words.astype(jnp.int32)
    neg_words = neg_words.astype(jnp.int32)
    idx = jnp.concatenate(
        [syn_words[:, None, :], neg_words], axis=1).reshape(B, ROWS)
    idx = jnp.concatenate(
        [idx, jnp.zeros((B, PADROWS - ROWS), jnp.int32)], axis=1)
    idx = idx.reshape(B, 2, CHUNK)

    mesh = plsc.VectorSubcoreMesh(
        core_axis_name="c", subcore_axis_name="s",
        num_cores=NC, num_subcores=NS)

    VOCAB = table.shape[0]
    tail = jnp.concatenate(
        [jnp.swapaxes(table[TBLK * 128:], 0, 1),
         jnp.zeros((D, 128 - TAILW), jnp.float32)], axis=1)
    flat = pl.kernel(
        _sc_transpose,
        out_type=jax.ShapeDtypeStruct((VOCAB * D,), jnp.float32),
        mesh=mesh,
        scratch_types=[
            pltpu.VMEM((D, 128), jnp.float32),
            pltpu.VMEM((D, 128), jnp.float32),
            pltpu.VMEM((128 * D,), jnp.float32),
            pltpu.VMEM((128 * D,), jnp.float32),
            pltpu.SemaphoreType.DMA((2,)),
            pltpu.SemaphoreType.DMA((2,)),
        ],
        compiler_params=pltpu.CompilerParams(needs_layout_passes=False),
    )(jnp.swapaxes(table, 0, 1), tail)
    table = flat.reshape(VOCAB, D)

    sc_out = pl.kernel(
        _sc_body,
        out_type=jax.ShapeDtypeStruct((13, B, D), jnp.float32),
        mesh=mesh,
        scratch_types=[
            pltpu.VMEM((BPW, 2, CHUNK), jnp.int32),
            pltpu.VMEM((2, PADROWS, D), jnp.float32),
            pltpu.VMEM((13, BPW, D), jnp.float32),
            pltpu.SemaphoreType.DMA,
            pltpu.SemaphoreType.DMA,
        ],
        compiler_params=pltpu.CompilerParams(use_tc_tiling_on_sc=False),
    )(idx, table)

    R = 512
    loss = pl.pallas_call(
        _tc_finish,
        grid=(B // R,),
        in_specs=[
            pl.BlockSpec((13, R, D), lambda i: (0, i, 0)),
            pl.BlockSpec((R, LW), lambda i: (i, 0)),
            pl.BlockSpec((R, N), lambda i: (i, 0)),
        ],
        out_specs=pl.BlockSpec((1, 1), lambda i: (0, 0)),
        out_shape=jax.ShapeDtypeStruct((1, 1), jnp.float32),
    )(sc_out, syn_words, margins)
    return loss[0, 0]
